# baseline (speedup 1.0000x reference)
import functools

import jax
import jax.numpy as jnp
from jax import lax
from jax.experimental import pallas as pl
from jax.experimental.pallas import tpu as pltpu
from jax.experimental.pallas import tpu_sc as plsc

N = 10000
E = 160000
H = 512
KPAD = 1536
NTILES = 32
EPT = E // NTILES
EB = 40
NCH = EPT // EB
AB = 128
ANCH = 40
NS = N + 16
RPT = 624
BN = 400


def _sc_mesh():
    return plsc.VectorSubcoreMesh(core_axis_name="c", subcore_axis_name="s")


def _make_deg():
    @functools.partial(
        pl.kernel,
        out_type=jax.ShapeDtypeStruct((2, N, 128), jnp.float32),
        mesh=_sc_mesh(),
        scratch_types=[
            pltpu.VMEM((NCH, EB), jnp.int32),
            pltpu.VMEM((EB, 128), jnp.float32),
            pltpu.VMEM((48, 128), jnp.float32),
            pltpu.VMEM_SHARED((N, 128), jnp.float32),
        ],
    )
    def deg_kernel(dst_hbm, out_hbm, dst_v, ones_v, zrow_v, shared):
        c = lax.axis_index("c")
        s = lax.axis_index("s")
        w = c * 16 + s
        base = s * RPT
        for i in range(EB):
            for q in range(8):
                ones_v[i, pl.ds(q * 16, 16)] = jnp.full((16,), 1.0, jnp.float32)
        for i in range(48):
            for q in range(8):
                zrow_v[i, pl.ds(q * 16, 16)] = jnp.zeros((16,), jnp.float32)
        for k in range(RPT // 48):
            pltpu.sync_copy(zrow_v, shared.at[pl.ds(base + k * 48, 48)])

        @pl.when(s == 15)
        def _():
            pltpu.sync_copy(zrow_v.at[pl.ds(0, 16)], shared.at[pl.ds(N - 16, 16)])

        plsc.subcore_barrier()
        pltpu.sync_copy(dst_hbm.at[w], dst_v)

        def body(j, carry):
            pltpu.sync_copy(ones_v, shared.at[dst_v.at[j]], add=True)
            return carry

        lax.fori_loop(0, NCH, body, 0)
        plsc.subcore_barrier()
        pltpu.sync_copy(shared.at[pl.ds(base, RPT)], out_hbm.at[c, pl.ds(base, RPT)])

        @pl.when(s == 15)
        def _():
            pltpu.sync_copy(shared.at[pl.ds(N - 16, 16)],
                            out_hbm.at[c, pl.ds(N - 16, 16)])

    return deg_kernel


def _make_agg(ncc, d):
    @functools.partial(
        pl.kernel,
        out_type=jax.ShapeDtypeStruct((2, ncc, N, d), jnp.float32),
        mesh=_sc_mesh(),
        scratch_types=[
            pltpu.VMEM((ANCH, AB), jnp.int32),
            pltpu.VMEM((ANCH, AB), jnp.int32),
            pltpu.VMEM((AB, d), jnp.float32),
            pltpu.VMEM((AB, d), jnp.float32),
            pltpu.VMEM_SHARED((NS, d), jnp.float32),
            pltpu.SemaphoreType.DMA,
            pltpu.SemaphoreType.DMA,
        ],
    )
    def agg_kernel(table_hbm, src_hbm, dst_hbm, out_hbm,
                   src_v, dst_v, rows0, rows1, shared, sem0, sem1):
        c = lax.axis_index("c")
        s = lax.axis_index("s")
        w = c * 16 + s
        base = s * RPT
        pltpu.sync_copy(src_hbm.at[w], src_v)
        pltpu.sync_copy(dst_hbm.at[w], dst_v)
        for cc in range(ncc):
            tab = table_hbm.at[cc]
            pltpu.sync_copy(tab.at[pl.ds(base, RPT)], shared.at[pl.ds(base, RPT)])

            @pl.when(s == 15)
            def _():
                pltpu.sync_copy(tab.at[pl.ds(N - 16, 16)],
                                shared.at[pl.ds(N - 16, 16)])

            plsc.subcore_barrier()

            pltpu.async_copy(tab.at[src_v.at[0]], rows0, sem0)

            def body(jj, carry):
                j0 = 2 * jj
                j1 = j0 + 1
                j2 = jnp.minimum(j0 + 2, ANCH - 1)
                pltpu.make_async_copy(tab.at[src_v.at[j0]], rows0, sem0).wait()
                pltpu.async_copy(tab.at[src_v.at[j1]], rows1, sem1)
                pltpu.sync_copy(rows0, shared.at[dst_v.at[j0]], add=True)
                pltpu.make_async_copy(tab.at[src_v.at[j1]], rows1, sem1).wait()
                pltpu.async_copy(tab.at[src_v.at[j2]], rows0, sem0)
                pltpu.sync_copy(rows1, shared.at[dst_v.at[j1]], add=True)
                return carry

            lax.fori_loop(0, ANCH // 2, body, 0)
            pltpu.make_async_copy(tab.at[src_v.at[ANCH - 1]], rows0, sem0).wait()
            plsc.subcore_barrier()
            pltpu.sync_copy(shared.at[pl.ds(base, RPT)],
                            out_hbm.at[c, cc, pl.ds(base, RPT)])

            @pl.when(s == 15)
            def _():
                pltpu.sync_copy(shared.at[pl.ds(N - 16, 16)],
                                out_hbm.at[c, cc, pl.ds(N - 16, 16)])

            plsc.subcore_barrier()

    return agg_kernel


def _dis(deg_blk):
    d = deg_blk[0, :, 0:1] + deg_blk[1, :, 0:1] + 1.0
    return lax.rsqrt(d)


def _mm1_body(x_ref, w_ref, deg_ref, out_ref):
    dis = _dis(deg_ref[...])
    h = jnp.dot(x_ref[...], w_ref[...], preferred_element_type=jnp.float32)
    h = h * dis
    for cdx in range(4):
        out_ref[cdx] = h[:, cdx * 128:(cdx + 1) * 128]


def _prologue(a_ref, h_ref, b_ref, dis):
    zs = []
    bfull = b_ref[...]
    for cdx in range(4):
        ac = a_ref[0, cdx] + a_ref[1, cdx] - h_ref[cdx]
        zs.append(jnp.maximum(ac * dis + bfull[0, cdx * 128:(cdx + 1) * 128], 0.0))
    return jnp.concatenate(zs, axis=1)


def _mm2_body(a_ref, h_ref, deg_ref, w_ref, b_ref, out_ref):
    dis = _dis(deg_ref[...])
    z = _prologue(a_ref, h_ref, b_ref, dis)
    h = jnp.dot(z, w_ref[...], preferred_element_type=jnp.float32)
    h = h * dis
    for cdx in range(4):
        out_ref[cdx] = h[:, cdx * 128:(cdx + 1) * 128]


def _mm3_body(a_ref, h_ref, deg_ref, w_ref, b_ref, out_ref):
    dis = _dis(deg_ref[...])
    z = _prologue(a_ref, h_ref, b_ref, dis)
    h = jnp.dot(z, w_ref[...], preferred_element_type=jnp.float32)
    out_ref[...] = h * dis


def _final_body(a_ref, h_ref, deg_ref, b_ref, out_ref):
    dis = _dis(deg_ref[...])
    a = a_ref[0] + a_ref[1] - h_ref[...]
    z = a * dis + b_ref[...][0]
    col = lax.broadcasted_iota(jnp.int32, z.shape, 1)
    z = jnp.where(col < 7, z, -1e30)
    m = jnp.max(z, axis=1, keepdims=True)
    zz = z - m
    lse = jnp.log(jnp.sum(jnp.exp(zz), axis=1, keepdims=True))
    out_ref[...] = zz - lse


def _blk(shape, index_map):
    return pl.BlockSpec(shape, index_map)


def kernel(x, edge_index, W1, b1, W2, b2, W3, b3):
    f_in = x.shape[1]
    w3p = jnp.pad(W3, ((0, 0), (0, 128 - W3.shape[1])))
    b3p = jnp.pad(b3, (0, 128 - b3.shape[0])).reshape(1, 128)
    b1r = b1.reshape(1, H)
    b2r = b2.reshape(1, H)
    dstr40 = edge_index[1].reshape(NTILES, NCH, EB)
    src2 = jnp.pad(edge_index[0].reshape(NTILES, EPT), ((0, 0), (0, 120)))
    junk = (N + jnp.arange(NTILES, dtype=jnp.int32) % 16)[:, None]
    dst2 = jnp.concatenate(
        [edge_index[1].reshape(NTILES, EPT),
         jnp.broadcast_to(junk, (NTILES, 120))], axis=1)
    srcr = src2.reshape(NTILES, ANCH, AB)
    dstr = dst2.reshape(NTILES, ANCH, AB)

    deg2 = _make_deg()(dstr40)

    grid = (N // BN,)
    mm1 = pl.pallas_call(
        _mm1_body,
        grid=grid,
        in_specs=[
            _blk((BN, f_in), lambda i: (i, 0)),
            _blk((f_in, H), lambda i: (0, 0)),
            _blk((2, BN, 128), lambda i: (0, i, 0)),
        ],
        out_specs=_blk((4, BN, 128), lambda i: (0, i, 0)),
        out_shape=jax.ShapeDtypeStruct((4, N, 128), jnp.float32),
    )
    h1 = mm1(x, W1, deg2)

    agg_wide = _make_agg(4, 128)
    a1p = agg_wide(h1, srcr, dstr)

    mm_mid_specs = dict(
        grid=grid,
        in_specs=[
            _blk((2, 4, BN, 128), lambda i: (0, 0, i, 0)),
            _blk((4, BN, 128), lambda i: (0, i, 0)),
            _blk((2, BN, 128), lambda i: (0, i, 0)),
            _blk((H, H), lambda i: (0, 0)),
            _blk((1, H), lambda i: (0, 0)),
        ],
    )
    mm2 = pl.pallas_call(
        _mm2_body,
        out_specs=_blk((4, BN, 128), lambda i: (0, i, 0)),
        out_shape=jax.ShapeDtypeStruct((4, N, 128), jnp.float32),
        **mm_mid_specs,
    )
    h2 = mm2(a1p, h1, deg2, W2, b1r)

    a2p = agg_wide(h2, srcr, dstr)

    mm3 = pl.pallas_call(
        _mm3_body,
        grid=grid,
        in_specs=[
            _blk((2, 4, BN, 128), lambda i: (0, 0, i, 0)),
            _blk((4, BN, 128), lambda i: (0, i, 0)),
            _blk((2, BN, 128), lambda i: (0, i, 0)),
            _blk((H, 128), lambda i: (0, 0)),
            _blk((1, H), lambda i: (0, 0)),
        ],
        out_specs=_blk((BN, 128), lambda i: (i, 0)),
        out_shape=jax.ShapeDtypeStruct((N, 128), jnp.float32),
    )
    h3 = mm3(a2p, h2, deg2, w3p, b2r)

    h3r = h3.reshape(1, N, 128)
    a3p = _make_agg(1, 128)(h3r, srcr, dstr)
    a3p = a3p.reshape(2, N, 128)

    final = pl.pallas_call(
        _final_body,
        grid=grid,
        in_specs=[
            _blk((2, BN, 128), lambda i: (0, i, 0)),
            _blk((BN, 128), lambda i: (i, 0)),
            _blk((2, BN, 128), lambda i: (0, i, 0)),
            _blk((1, 128), lambda i: (0, 0)),
        ],
        out_specs=_blk((BN, 128), lambda i: (i, 0)),
        out_shape=jax.ShapeDtypeStruct((N, 128), jnp.float32),
    )
    out = final(a3p, h3, deg2, b3p)
    return out[:, :7]


# speedup vs baseline: 3.9403x; 3.9403x over previous
import functools

import jax
import jax.numpy as jnp
from jax import lax
from jax.experimental import pallas as pl
from jax.experimental.pallas import tpu as pltpu
from jax.experimental.pallas import tpu_sc as plsc

N = 10000
E = 160000
H = 512
KPAD = 1536
NTILES = 32
EPT = E // NTILES
EB = 40
NCH = EPT // EB
AB = 128
ANCH = 39
ATAIL = EPT - ANCH * AB
RPT = 624
BN = 400


def _sc_mesh():
    return plsc.VectorSubcoreMesh(core_axis_name="c", subcore_axis_name="s")


def _make_deg():
    @functools.partial(
        pl.kernel,
        out_type=jax.ShapeDtypeStruct((2, N, 128), jnp.float32),
        mesh=_sc_mesh(),
        scratch_types=[
            pltpu.VMEM((NCH, EB), jnp.int32),
            pltpu.VMEM((EB, 128), jnp.float32),
            pltpu.VMEM((48, 128), jnp.float32),
            pltpu.VMEM_SHARED((N, 128), jnp.float32),
        ],
    )
    def deg_kernel(dst_hbm, out_hbm, dst_v, ones_v, zrow_v, shared):
        c = lax.axis_index("c")
        s = lax.axis_index("s")
        w = c * 16 + s
        base = s * RPT
        for i in range(EB):
            for q in range(8):
                ones_v[i, pl.ds(q * 16, 16)] = jnp.full((16,), 1.0, jnp.float32)
        for i in range(48):
            for q in range(8):
                zrow_v[i, pl.ds(q * 16, 16)] = jnp.zeros((16,), jnp.float32)
        for k in range(RPT // 48):
            pltpu.sync_copy(zrow_v, shared.at[pl.ds(base + k * 48, 48)])

        @pl.when(s == 15)
        def _():
            pltpu.sync_copy(zrow_v.at[pl.ds(0, 16)], shared.at[pl.ds(N - 16, 16)])

        plsc.subcore_barrier()
        pltpu.sync_copy(dst_hbm.at[w], dst_v)

        def body(j, carry):
            pltpu.sync_copy(ones_v, shared.at[dst_v.at[j]], add=True)
            return carry

        lax.fori_loop(0, NCH, body, 0)
        plsc.subcore_barrier()
        pltpu.sync_copy(shared.at[pl.ds(base, RPT)], out_hbm.at[c, pl.ds(base, RPT)])

        @pl.when(s == 15)
        def _():
            pltpu.sync_copy(shared.at[pl.ds(N - 16, 16)],
                            out_hbm.at[c, pl.ds(N - 16, 16)])

    return deg_kernel


def _make_agg(ncc, d):
    @functools.partial(
        pl.kernel,
        out_type=jax.ShapeDtypeStruct((2, ncc, N, d), jnp.float32),
        mesh=_sc_mesh(),
        scratch_types=[
            pltpu.VMEM((ANCH, AB), jnp.int32),
            pltpu.VMEM((ANCH, AB), jnp.int32),
            pltpu.VMEM((1, ATAIL), jnp.int32),
            pltpu.VMEM((1, ATAIL), jnp.int32),
            pltpu.VMEM((AB, d), jnp.float32),
            pltpu.VMEM((AB, d), jnp.float32),
            pltpu.VMEM_SHARED((N, d), jnp.float32),
            pltpu.SemaphoreType.DMA,
            pltpu.SemaphoreType.DMA,
        ],
    )
    def agg_kernel(table_hbm, src_hbm, dst_hbm, tsrc_hbm, tdst_hbm, out_hbm,
                   src_v, dst_v, tsrc_v, tdst_v, rows0, rows1, shared,
                   sem0, sem1):
        c = lax.axis_index("c")
        s = lax.axis_index("s")
        w = c * 16 + s
        base = s * RPT
        pltpu.sync_copy(src_hbm.at[w], src_v)
        pltpu.sync_copy(dst_hbm.at[w], dst_v)
        pltpu.sync_copy(tsrc_hbm.at[pl.ds(w, 1)], tsrc_v)
        pltpu.sync_copy(tdst_hbm.at[pl.ds(w, 1)], tdst_v)
        for cc in range(ncc):
            tab = table_hbm.at[cc]
            pltpu.sync_copy(tab.at[pl.ds(base, RPT)], shared.at[pl.ds(base, RPT)])

            @pl.when(s == 15)
            def _():
                pltpu.sync_copy(tab.at[pl.ds(N - 16, 16)],
                                shared.at[pl.ds(N - 16, 16)])

            plsc.subcore_barrier()

            pltpu.async_copy(tab.at[src_v.at[0]], rows0, sem0)

            def body(jj, carry):
                j0 = 2 * jj
                j1 = j0 + 1
                j2 = j0 + 2
                pltpu.make_async_copy(tab.at[src_v.at[j0]], rows0, sem0).wait()
                pltpu.async_copy(tab.at[src_v.at[j1]], rows1, sem1)
                pltpu.sync_copy(rows0, shared.at[dst_v.at[j0]], add=True)
                pltpu.make_async_copy(tab.at[src_v.at[j1]], rows1, sem1).wait()
                pltpu.async_copy(tab.at[src_v.at[j2]], rows0, sem0)
                pltpu.sync_copy(rows1, shared.at[dst_v.at[j1]], add=True)
                return carry

            lax.fori_loop(0, (ANCH - 1) // 2, body, 0)
            pltpu.make_async_copy(tab.at[src_v.at[ANCH - 1]], rows0, sem0).wait()
            pltpu.sync_copy(rows0, shared.at[dst_v.at[ANCH - 1]], add=True)
            tailbuf = rows0.at[pl.ds(0, ATAIL)]
            pltpu.sync_copy(tab.at[tsrc_v.at[0]], tailbuf)
            pltpu.sync_copy(tailbuf, shared.at[tdst_v.at[0]], add=True)
            plsc.subcore_barrier()
            pltpu.sync_copy(shared.at[pl.ds(base, RPT)],
                            out_hbm.at[c, cc, pl.ds(base, RPT)])

            @pl.when(s == 15)
            def _():
                pltpu.sync_copy(shared.at[pl.ds(N - 16, 16)],
                                out_hbm.at[c, cc, pl.ds(N - 16, 16)])

            plsc.subcore_barrier()

    return agg_kernel


def _dis(deg_blk):
    d = deg_blk[0, :, 0:1] + deg_blk[1, :, 0:1] + 1.0
    return lax.rsqrt(d)


def _mm1_body(x_ref, w_ref, deg_ref, out_ref):
    dis = _dis(deg_ref[...])
    h = jnp.dot(x_ref[...], w_ref[...], preferred_element_type=jnp.float32)
    h = h * dis
    for cdx in range(4):
        out_ref[cdx] = h[:, cdx * 128:(cdx + 1) * 128]


def _prologue(a_ref, h_ref, b_ref, dis):
    zs = []
    bfull = b_ref[...]
    for cdx in range(4):
        ac = a_ref[0, cdx] + a_ref[1, cdx] - h_ref[cdx]
        zs.append(jnp.maximum(ac * dis + bfull[0, cdx * 128:(cdx + 1) * 128], 0.0))
    return jnp.concatenate(zs, axis=1)


def _mm2_body(a_ref, h_ref, deg_ref, w_ref, b_ref, out_ref):
    dis = _dis(deg_ref[...])
    z = _prologue(a_ref, h_ref, b_ref, dis)
    h = jnp.dot(z, w_ref[...], preferred_element_type=jnp.float32)
    h = h * dis
    for cdx in range(4):
        out_ref[cdx] = h[:, cdx * 128:(cdx + 1) * 128]


def _mm3_body(a_ref, h_ref, deg_ref, w_ref, b_ref, out_ref):
    dis = _dis(deg_ref[...])
    z = _prologue(a_ref, h_ref, b_ref, dis)
    h = jnp.dot(z, w_ref[...], preferred_element_type=jnp.float32)
    out_ref[...] = h * dis


def _final_body(a_ref, h_ref, deg_ref, b_ref, out_ref):
    dis = _dis(deg_ref[...])
    a = a_ref[0] + a_ref[1] - h_ref[...]
    z = a * dis + b_ref[...][0]
    col = lax.broadcasted_iota(jnp.int32, z.shape, 1)
    z = jnp.where(col < 7, z, -1e30)
    m = jnp.max(z, axis=1, keepdims=True)
    zz = z - m
    lse = jnp.log(jnp.sum(jnp.exp(zz), axis=1, keepdims=True))
    out_ref[...] = zz - lse


def _blk(shape, index_map):
    return pl.BlockSpec(shape, index_map)


def kernel(x, edge_index, W1, b1, W2, b2, W3, b3):
    f_in = x.shape[1]
    w3p = jnp.pad(W3, ((0, 0), (0, 128 - W3.shape[1])))
    b3p = jnp.pad(b3, (0, 128 - b3.shape[0])).reshape(1, 128)
    b1r = b1.reshape(1, H)
    b2r = b2.reshape(1, H)
    dstr40 = edge_index[1].reshape(NTILES, NCH, EB)
    e0 = edge_index[0].reshape(NTILES, EPT)
    e1 = edge_index[1].reshape(NTILES, EPT)
    srcr = e0[:, :ANCH * AB].reshape(NTILES, ANCH, AB)
    dstr = e1[:, :ANCH * AB].reshape(NTILES, ANCH, AB)
    tsrc = e0[:, ANCH * AB:]
    tdst = e1[:, ANCH * AB:]

    deg2 = _make_deg()(dstr40)

    grid = (N // BN,)
    mm1 = pl.pallas_call(
        _mm1_body,
        grid=grid,
        in_specs=[
            _blk((BN, f_in), lambda i: (i, 0)),
            _blk((f_in, H), lambda i: (0, 0)),
            _blk((2, BN, 128), lambda i: (0, i, 0)),
        ],
        out_specs=_blk((4, BN, 128), lambda i: (0, i, 0)),
        out_shape=jax.ShapeDtypeStruct((4, N, 128), jnp.float32),
    )
    h1 = mm1(x, W1, deg2)

    agg_wide = _make_agg(4, 128)
    a1p = agg_wide(h1, srcr, dstr, tsrc, tdst)

    mm_mid_specs = dict(
        grid=grid,
        in_specs=[
            _blk((2, 4, BN, 128), lambda i: (0, 0, i, 0)),
            _blk((4, BN, 128), lambda i: (0, i, 0)),
            _blk((2, BN, 128), lambda i: (0, i, 0)),
            _blk((H, H), lambda i: (0, 0)),
            _blk((1, H), lambda i: (0, 0)),
        ],
    )
    mm2 = pl.pallas_call(
        _mm2_body,
        out_specs=_blk((4, BN, 128), lambda i: (0, i, 0)),
        out_shape=jax.ShapeDtypeStruct((4, N, 128), jnp.float32),
        **mm_mid_specs,
    )
    h2 = mm2(a1p, h1, deg2, W2, b1r)

    a2p = agg_wide(h2, srcr, dstr, tsrc, tdst)

    mm3 = pl.pallas_call(
        _mm3_body,
        grid=grid,
        in_specs=[
            _blk((2, 4, BN, 128), lambda i: (0, 0, i, 0)),
            _blk((4, BN, 128), lambda i: (0, i, 0)),
            _blk((2, BN, 128), lambda i: (0, i, 0)),
            _blk((H, 128), lambda i: (0, 0)),
            _blk((1, H), lambda i: (0, 0)),
        ],
        out_specs=_blk((BN, 128), lambda i: (i, 0)),
        out_shape=jax.ShapeDtypeStruct((N, 128), jnp.float32),
    )
    h3 = mm3(a2p, h2, deg2, w3p, b2r)

    h3r = h3.reshape(1, N, 128)
    a3p = _make_agg(1, 128)(h3r, srcr, dstr, tsrc, tdst)
    a3p = a3p.reshape(2, N, 128)

    final = pl.pallas_call(
        _final_body,
        grid=grid,
        in_specs=[
            _blk((2, BN, 128), lambda i: (0, i, 0)),
            _blk((BN, 128), lambda i: (i, 0)),
            _blk((2, BN, 128), lambda i: (0, i, 0)),
            _blk((1, 128), lambda i: (0, 0)),
        ],
        out_specs=_blk((BN, 128), lambda i: (i, 0)),
        out_shape=jax.ShapeDtypeStruct((N, 128), jnp.float32),
    )
    out = final(a3p, h3, deg2, b3p)
    return out[:, :7]
